# Initial kernel scaffold; baseline (speedup 1.0000x reference)
import jax
import jax.numpy as jnp
from jax.experimental import pallas as pl


def kernel(x, rbf, idx_i, idx_j, W_rbf, W_edge, b_edge):
    raise NotImplementedError("write your pallas kernel here")


# speedup vs baseline: 3.3222x; 3.3222x over previous
import functools

import jax
import jax.numpy as jnp
from jax import lax
from jax.experimental import pallas as pl
from jax.experimental.pallas import tpu as pltpu
from jax.experimental.pallas import tpu_sc as plsc

NC = 2
NS = 16
NW = NC * NS

CHUNK = 400


def _tables_body(x_ref, w1_ref, w2_ref, wrbf_ref, w3_ref, y_ref, z_ref, w3c_ref):
    x = x_ref[...]
    y_ref[...] = jnp.dot(x, w1_ref[...], preferred_element_type=jnp.float32)
    z_ref[...] = jnp.dot(x, w2_ref[...], preferred_element_type=jnp.float32)
    w3c_ref[...] = jnp.dot(wrbf_ref[...], w3_ref[...],
                           preferred_element_type=jnp.float32)


def _make_tables(x, w1, w2, wrbf, w3):
    n, d = x.shape
    return pl.pallas_call(
        _tables_body,
        out_shape=(
            jax.ShapeDtypeStruct((n, d), jnp.float32),
            jax.ShapeDtypeStruct((n, d), jnp.float32),
            jax.ShapeDtypeStruct((wrbf.shape[0], d), jnp.float32),
        ),
    )(x, w1, w2, wrbf, w3)


def _sc_gather_body(epw, y_hbm, z_hbm, idxj_hbm, idxi_hbm, g_hbm, h_hbm,
                    idxj_v, idxi_v, yj_v, zi_v, sem1, sem2):
    wid = lax.axis_index("s") * NC + lax.axis_index("c")
    base0 = wid * epw

    def chunk(t, carry):
        base = base0 + t * CHUNK
        pltpu.sync_copy(idxj_hbm.at[pl.ds(base, CHUNK)], idxj_v)
        pltpu.sync_copy(idxi_hbm.at[pl.ds(base, CHUNK)], idxi_v)
        cp1 = pltpu.async_copy(y_hbm.at[idxj_v], yj_v, sem1)
        cp2 = pltpu.async_copy(z_hbm.at[idxi_v], zi_v, sem2)
        cp1.wait()
        cp2.wait()
        pltpu.sync_copy(yj_v, g_hbm.at[pl.ds(base, CHUNK)])
        pltpu.sync_copy(zi_v, h_hbm.at[pl.ds(base, CHUNK)])
        return carry

    lax.fori_loop(0, epw // CHUNK, chunk, 0)


def _sc_gather(y, z, idx_j, idx_i):
    e = idx_j.shape[0]
    d = y.shape[1]
    assert e % (NW * CHUNK) == 0
    epw = e // NW
    mesh = plsc.VectorSubcoreMesh(core_axis_name="c", subcore_axis_name="s",
                                  num_cores=NC, num_subcores=NS)
    kern = pl.kernel(
        functools.partial(_sc_gather_body, epw),
        out_type=(
            jax.ShapeDtypeStruct((e, d), jnp.float32),
            jax.ShapeDtypeStruct((e, d), jnp.float32),
        ),
        mesh=mesh,
        scratch_types=[
            pltpu.VMEM((CHUNK,), jnp.int32),
            pltpu.VMEM((CHUNK,), jnp.int32),
            pltpu.VMEM((CHUNK, d), jnp.float32),
            pltpu.VMEM((CHUNK, d), jnp.float32),
            pltpu.SemaphoreType.DMA,
            pltpu.SemaphoreType.DMA,
        ],
    )
    return kern(y, z, idx_j, idx_i)


def _combine_body(g_ref, h_ref, rbf_ref, w3c_ref, b_ref, out_ref):
    s = (g_ref[...] + h_ref[...]
         + jnp.dot(rbf_ref[...], w3c_ref[...],
                   preferred_element_type=jnp.float32)
         + b_ref[...])
    out_ref[...] = s * jax.nn.sigmoid(s)


def _combine(g, h, rbf, w3c, b2d, block):
    e, d = g.shape
    nrad = rbf.shape[1]
    assert e % block == 0
    grid = (e // block,)
    return pl.pallas_call(
        _combine_body,
        grid=grid,
        in_specs=[
            pl.BlockSpec((block, d), lambda i: (i, 0)),
            pl.BlockSpec((block, d), lambda i: (i, 0)),
            pl.BlockSpec((block, nrad), lambda i: (i, 0)),
            pl.BlockSpec((nrad, d), lambda i: (0, 0)),
            pl.BlockSpec((1, d), lambda i: (0, 0)),
        ],
        out_specs=pl.BlockSpec((block, d), lambda i: (i, 0)),
        out_shape=jax.ShapeDtypeStruct((e, d), jnp.float32),
    )(g, h, rbf, w3c, b2d)


def kernel(x, rbf, idx_i, idx_j, W_rbf, W_edge, b_edge):
    d = x.shape[1]
    w1 = W_edge[:d]
    w2 = W_edge[d:2 * d]
    w3 = W_edge[2 * d:]
    idx_i = idx_i.astype(jnp.int32)
    idx_j = idx_j.astype(jnp.int32)

    y, z, w3c = _make_tables(x, w1, w2, W_rbf, w3)
    g, h = _sc_gather(y, z, idx_j, idx_i)
    return _combine(g, h, rbf, w3c, b_edge.reshape(1, d), block=4000)
